# baseline (speedup 1.0000x reference)
import functools
import jax
import jax.numpy as jnp
from jax import lax
from jax.experimental import pallas as pl
from jax.experimental.pallas import tpu as pltpu, tpu_sc as plsc

NUM_GRAPHS = 16
NW = 32

_SC_PARAMS = pltpu.CompilerParams(
    needs_layout_passes=False, use_tc_tiling_on_sc=False)
_SC_MESH = plsc.VectorSubcoreMesh(core_axis_name="c", subcore_axis_name="s")


def _leaky(x):
  return jnp.where(x >= 0, x, 0.1 * x)


def _node_mlp_body(x3_ref, w1_ref, b1_ref, w2s_ref, b2_ref, o_ref):
  acc = None
  for s in range(8):
    h = jnp.dot(x3_ref[:, s, :], w1_ref[...],
                preferred_element_type=jnp.float32)
    h = _leaky(h + b1_ref[...])
    y = jnp.dot(h, w2s_ref[s], preferred_element_type=jnp.float32)
    acc = y if acc is None else acc + y
  o_ref[...] = acc + b2_ref[...]


def _node_mlp(x3, Wn1, bn1, Wn2s, bn2r, npad, blkg):
  ng = npad // 8
  grid = ng // blkg
  return pl.pallas_call(
      _node_mlp_body,
      grid=(grid,),
      in_specs=[
          pl.BlockSpec((blkg, 8, 128), lambda i: (i, 0, 0)),
          pl.BlockSpec((128, 128), lambda i: (0, 0)),
          pl.BlockSpec((1, 128), lambda i: (0, 0)),
          pl.BlockSpec((8, 128, 128), lambda i: (0, 0, 0)),
          pl.BlockSpec((1, 128), lambda i: (0, 0)),
      ],
      out_specs=pl.BlockSpec((blkg, 128), lambda i: (i, 0)),
      out_shape=jax.ShapeDtypeStruct((ng, 128), jnp.float32),
  )(x3, Wn1, bn1, Wn2s, bn2r)


def _make_first3_local(E, npad):
  ew = E // NW
  tcols = E // 128
  ntl = ew // 128 + 2

  def body(ei_hbm, cnt_hbm, slots_hbm, ei_v, cnt_v, slots_v):
    cid = lax.axis_index("c")
    sid = lax.axis_index("s")
    w = sid * 2 + cid
    a = w * ew
    ta = jnp.minimum(a // 128, tcols - ntl)
    pltpu.sync_copy(ei_hbm.at[pl.ds(ta * 256, ntl * 256)], ei_v)

    def zero_body(i, _):
      cnt_v[pl.ds(i * 16, 16)] = jnp.zeros((16,), jnp.int32)
      return 0
    lax.fori_loop(0, npad // 16, zero_body, 0)

    iota = lax.iota(jnp.int32, 16)

    def scan_body(kv, _):
      pos = kv * 16 + (kv // 8) * 128
      s = ei_v[pl.ds(pos, 16)]
      eid = ta * 128 + kv * 16 + iota
      valid = (eid >= a) & (eid < a + ew)
      cnt1, last = plsc.scan_count(s, mask=valid)
      prior = plsc.load_gather(cnt_v, [s])
      r = prior + cnt1 - 1
      slot = s * 3 + jnp.minimum(jnp.maximum(r, 0), 2)
      plsc.store_scatter(slots_v, [slot], eid, mask=valid & (r < 3))
      plsc.store_scatter(cnt_v, [s], prior + cnt1, mask=last & valid)
      return 0
    lax.fori_loop(0, ntl * 8, scan_body, 0)

    pltpu.sync_copy(cnt_v, cnt_hbm.at[w])
    pltpu.sync_copy(slots_v, slots_hbm.at[w])

  return pl.kernel(
      body,
      out_type=(jax.ShapeDtypeStruct((NW, npad), jnp.int32),
                jax.ShapeDtypeStruct((NW, 3 * npad), jnp.int32)),
      mesh=_SC_MESH,
      compiler_params=_SC_PARAMS,
      scratch_types=[
          pltpu.VMEM((ntl * 256,), jnp.int32),
          pltpu.VMEM((npad,), jnp.int32),
          pltpu.VMEM((3 * npad,), jnp.int32),
      ])


def _make_merge_gather(E, npad):
  npw = npad // NW
  tcols = E // 128

  def body(cnt_hbm, slots_hbm, eif_hbm, ea_hbm, nf_hbm,
           packed_out, cnt_out,
           loc_cnt, loc_slots, cm_v, m0_v, m1_v, m2_v, nbr_v, rows_v,
           pk_v, idx_v, ev_v, sem):
    cid = lax.axis_index("c")
    sid = lax.axis_index("s")
    wid = sid * 2 + cid
    nbase = wid * npw

    c1 = pltpu.async_copy(cnt_hbm.at[:, pl.ds(nbase, npw)], loc_cnt, sem)
    c2 = pltpu.async_copy(slots_hbm.at[:, pl.ds(3 * nbase, 3 * npw)],
                          loc_slots, sem)
    c1.wait()
    c2.wait()

    iota = lax.iota(jnp.int32, 16)
    zero = jnp.zeros((16,), jnp.int32)

    def merge_body(nv, _):
      nloc3 = (nv * 16 + iota) * 3
      cnt = zero
      mm0 = zero
      mm1 = zero
      mm2 = zero
      for w in range(NW):
        c = loc_cnt[w, pl.ds(nv * 16, 16)]
        wv = jnp.full((16,), w, jnp.int32)
        b0 = plsc.load_gather(loc_slots, [wv, nloc3])
        b1 = plsc.load_gather(loc_slots, [wv, nloc3 + 1])
        b2 = plsc.load_gather(loc_slots, [wv, nloc3 + 2])
        ce = jnp.minimum(c, 3)
        f0 = cnt == 0
        f1 = cnt == 1
        f2 = cnt == 2
        mm0 = jnp.where(f0 & (ce >= 1), b0, mm0)
        mm1 = jnp.where(f0 & (ce >= 2), b1,
                        jnp.where(f1 & (ce >= 1), b0, mm1))
        mm2 = jnp.where(f0 & (ce >= 3), b2,
                        jnp.where(f1 & (ce >= 2), b1,
                                  jnp.where(f2 & (ce >= 1), b0, mm2)))
        cnt = jnp.minimum(cnt + ce, 3)
      fb = nbase + nv * 16 + iota
      sl = pl.ds(nv * 16, 16)
      m0_v[sl] = jnp.where(cnt >= 1, mm0, fb)
      m1_v[sl] = jnp.where(cnt >= 2, mm1, fb)
      m2_v[sl] = jnp.where(cnt >= 3, mm2, fb)
      cm_v[sl] = cnt
      return 0
    lax.fori_loop(0, npw // 16, merge_body, 0)

    zeros16f = jnp.zeros((16,), jnp.float32)

    pltpu.sync_copy(nf_hbm.at[pl.ds(nbase, npw)], rows_v)

    def own_body(t, _):
      pk_v[t, pl.ds(96, 16)] = rows_v[t, pl.ds(0, 16)]
      pk_v[t, pl.ds(112, 16)] = zeros16f
      return 0
    lax.fori_loop(0, npw, own_body, 0)

    for j, m_v in enumerate((m0_v, m1_v, m2_v)):
      def gidx_body(t, _):
        m = m_v[pl.ds(t * 16, 16)]
        g = lax.shift_right_logical(m, 7) * 1024 + (m & 127)
        for c in range(16):
          fc = (c // 8) * (tcols * 1024) + (c % 8) * 128
          idx_v[pl.ds(c * npw + t * 16, 16)] = g + fc
        return 0
      lax.fori_loop(0, npw // 16, gidx_body, 0)
      pltpu.async_copy(ea_hbm.at[idx_v], ev_v, sem).wait()

      def esc_body(t, _):
        rows = t * 16 + iota
        for c in range(16):
          v = ev_v[pl.ds(c * npw + t * 16, 16)]
          plsc.store_scatter(pk_v, [rows, jnp.full((16,), j * 16 + c,
                                                   jnp.int32)], v)
        return 0
      lax.fori_loop(0, npw // 16, esc_body, 0)

      def didx_body(t, _):
        m = m_v[pl.ds(t * 16, 16)]
        nbr_v[pl.ds(t * 16, 16)] = (
            lax.shift_right_logical(m, 7) * 256 + 128 + (m & 127))
        return 0
      lax.fori_loop(0, npw // 16, didx_body, 0)
      pltpu.async_copy(eif_hbm.at[nbr_v], m_v, sem).wait()
      pltpu.async_copy(nf_hbm.at[m_v], rows_v, sem).wait()

      def nfc_body(t, _):
        pk_v[t, pl.ds(48 + j * 16, 16)] = rows_v[t, pl.ds(0, 16)]
        return 0
      lax.fori_loop(0, npw, nfc_body, 0)

    pltpu.sync_copy(pk_v, packed_out.at[pl.ds(nbase, npw)])
    pltpu.sync_copy(cm_v, cnt_out.at[pl.ds(nbase, npw)])

  return pl.kernel(
      body,
      out_type=(jax.ShapeDtypeStruct((npad, 128), jnp.float32),
                jax.ShapeDtypeStruct((npad,), jnp.int32)),
      mesh=_SC_MESH,
      compiler_params=_SC_PARAMS,
      scratch_types=[
          pltpu.VMEM((NW, npw), jnp.int32),
          pltpu.VMEM((NW, 3 * npw), jnp.int32),
          pltpu.VMEM((npw,), jnp.int32),
          pltpu.VMEM((npw,), jnp.int32),
          pltpu.VMEM((npw,), jnp.int32),
          pltpu.VMEM((npw,), jnp.int32),
          pltpu.VMEM((npw,), jnp.int32),
          pltpu.VMEM((npw, 16), jnp.float32),
          pltpu.VMEM((npw, 128), jnp.float32),
          pltpu.VMEM((16 * npw,), jnp.int32),
          pltpu.VMEM((16 * npw,), jnp.float32),
          pltpu.SemaphoreType.DMA,
      ])


def _tail_body(pk_ref, cnt_ref, bat_ref,
               We1j_ref, be1_ref, We2p_ref, be2p_ref,
               Wm1ap_ref, Wm1bj_ref, bm1_ref, Wm2p_ref, bm2p3_ref,
               S96_ref, Wu1c_ref, Wu1bp_ref, bu1_ref, Wu2p_ref, bu2p_ref,
               Wh1p_ref, bh1_ref, Wh2_ref, bh2_ref,
               o_ref, acc_ref):
  i = pl.program_id(0)
  n = pl.num_programs(0)

  pk = pk_ref[...]
  s_msg = None
  for j in range(3):
    he = _leaky(jnp.dot(pk, We1j_ref[j],
                        preferred_element_type=jnp.float32) + be1_ref[...])
    ef = jnp.dot(he, We2p_ref[...],
                 preferred_element_type=jnp.float32) + be2p_ref[...]
    pre = (jnp.dot(ef, Wm1ap_ref[...], preferred_element_type=jnp.float32)
           + jnp.dot(pk, Wm1bj_ref[j],
                     preferred_element_type=jnp.float32) + bm1_ref[...])
    lj = _leaky(pre)
    s_msg = lj if s_msg is None else s_msg + lj
  aggr = jnp.dot(s_msg, Wm2p_ref[...],
                 preferred_element_type=jnp.float32) + bm2p3_ref[...]

  nf0 = jnp.dot(pk, S96_ref[...], preferred_element_type=jnp.float32)
  hu = _leaky(jnp.dot(pk, Wu1c_ref[...], preferred_element_type=jnp.float32)
              + jnp.dot(aggr, Wu1bp_ref[...],
                        preferred_element_type=jnp.float32) + bu1_ref[...])
  nc = jnp.dot(hu, Wu2p_ref[...],
               preferred_element_type=jnp.float32) + bu2p_ref[...]
  valid = cnt_ref[...] >= 3
  nf_final = nf0 + jnp.where(valid, nc, 0.0)

  gids = lax.broadcasted_iota(jnp.int32, (1, NUM_GRAPHS), 1)
  oh = (bat_ref[...] == gids).astype(jnp.float32)
  seg = lax.dot_general(oh, nf_final, (((0,), (0,)), ((), ())),
                        preferred_element_type=jnp.float32)

  @pl.when(i == 0)
  def _():
    acc_ref[...] = jnp.zeros_like(acc_ref)
  acc_ref[...] += seg

  @pl.when(i == n - 1)
  def _():
    hh = _leaky(jnp.dot(acc_ref[...], Wh1p_ref[...],
                        preferred_element_type=jnp.float32) + bh1_ref[...])
    o_ref[...] = jnp.dot(hh, Wh2_ref[...],
                         preferred_element_type=jnp.float32) + bh2_ref[...]


def _tail(packed, cnt2d, bat2d, weights, npad, blk):
  grid = npad // blk
  full = lambda shape: pl.BlockSpec(shape, lambda i: tuple(0 for _ in shape))
  in_specs = [
      pl.BlockSpec((blk, 128), lambda i: (i, 0)),
      pl.BlockSpec((blk, 1), lambda i: (i, 0)),
      pl.BlockSpec((blk, 1), lambda i: (i, 0)),
      full((3, 128, 128)), full((1, 128)), full((128, 16)), full((1, 16)),
      full((16, 128)), full((3, 128, 128)), full((1, 128)),
      full((128, 16)), full((1, 16)),
      full((128, 16)), full((128, 128)), full((16, 128)), full((1, 128)),
      full((128, 16)), full((1, 16)),
      full((16, 128)), full((1, 128)), full((128, 2)), full((1, 2)),
  ]
  return pl.pallas_call(
      _tail_body,
      grid=(grid,),
      in_specs=in_specs,
      out_specs=pl.BlockSpec((NUM_GRAPHS, 2), lambda i: (0, 0)),
      out_shape=jax.ShapeDtypeStruct((NUM_GRAPHS, 2), jnp.float32),
      scratch_shapes=[pltpu.VMEM((NUM_GRAPHS, 16), jnp.float32)],
  )(packed, cnt2d, bat2d, *weights)


def kernel(node_feat, edge_attr, edge_index, batch,
           Wn1, bn1, Wn2, bn2, We1, be1, We2, be2,
           Wm1, bm1, Wm2, bm2, Wu1, bu1, Wu2, bu2,
           Wh1, bh1, Wh2, bh2):
  N, DF = node_feat.shape
  E = edge_attr.shape[0]
  npad = ((N + NW * 16 - 1) // (NW * 16)) * (NW * 16)
  blk = 2048
  tcols = E // 128

  node_feat_pad = jnp.pad(node_feat.astype(jnp.float32),
                          ((0, npad - N), (0, 0)))
  x3 = node_feat_pad.reshape(npad // 8, 8, 128)
  bat2d = jnp.pad(batch.astype(jnp.int32), (0, npad - N),
                  constant_values=NUM_GRAPHS).reshape(npad, 1)

  ea_lin = (edge_attr.astype(jnp.float32).T
            .reshape(2, 8, tcols, 128)
            .transpose(0, 2, 1, 3)
            .reshape(E * 16))
  ei_flat = edge_index.reshape(2, tcols, 128).transpose(1, 0, 2).reshape(2 * E)

  f32 = jnp.float32
  Wn2s = jnp.stack(
      [jnp.zeros((128, 128), f32).at[:, s * 16:s * 16 + 3].set(Wn2)
       for s in range(8)])
  bn2r = jnp.tile(jnp.zeros((16,), f32).at[:3].set(bn2), 8).reshape(1, 128)
  We2p = jnp.zeros((128, 16), f32).at[:, :3].set(We2)
  be2p = jnp.zeros((1, 16), f32).at[0, :3].set(be2)
  Wm1ap = jnp.zeros((16, 128), f32).at[:3].set(Wm1[:3])
  We1j = jnp.stack([jnp.zeros((128, 128), f32).at[j * 16:j * 16 + 16].set(We1)
                    for j in range(3)])
  Wm1bj = jnp.stack(
      [jnp.zeros((128, 128), f32).at[48 + j * 16:48 + j * 16 + 3].set(Wm1[3:6])
       for j in range(3)])
  Wm2p = jnp.zeros((128, 16), f32).at[:, :2].set(Wm2)
  bm2p3 = jnp.zeros((1, 16), f32).at[0, :2].set(3.0 * bm2)
  S96 = jnp.zeros((128, 16), f32).at[96:112].set(jnp.eye(16, dtype=f32))
  Wu1c = jnp.zeros((128, 128), f32).at[96:99].set(Wu1[:3])
  Wu1bp = jnp.zeros((16, 128), f32).at[:2].set(Wu1[3:5])
  Wu2p = jnp.zeros((128, 16), f32).at[:, :3].set(Wu2)
  bu2p = jnp.zeros((1, 16), f32).at[0, :3].set(bu2)
  Wh1p = jnp.zeros((16, 128), f32).at[:3].set(Wh1)

  nf128 = _node_mlp(x3, Wn1, bn1.reshape(1, 128), Wn2s, bn2r, npad,
                    npad // 8 // 5)
  nf_lin = nf128.reshape(npad, 16)

  cnt_loc, slots_loc = _make_first3_local(E, npad)(ei_flat)
  packed, cntm = _make_merge_gather(E, npad)(
      cnt_loc, slots_loc, ei_flat, ea_lin, nf_lin)

  weights = (We1j, be1.reshape(1, 128), We2p, be2p,
             Wm1ap, Wm1bj, bm1.reshape(1, 128), Wm2p, bm2p3,
             S96, Wu1c, Wu1bp, bu1.reshape(1, 128), Wu2p, bu2p,
             Wh1p, bh1.reshape(1, 128), Wh2, bh2.reshape(1, 2))
  return _tail(packed, cntm.reshape(npad, 1), bat2d, weights, npad, blk)


# speedup vs baseline: 1.0269x; 1.0269x over previous
import functools
import jax
import jax.numpy as jnp
from jax import lax
from jax.experimental import pallas as pl
from jax.experimental.pallas import tpu as pltpu, tpu_sc as plsc

NUM_GRAPHS = 16
NW = 32

_SC_PARAMS = pltpu.CompilerParams(
    needs_layout_passes=False, use_tc_tiling_on_sc=False)
_SC_MESH = plsc.VectorSubcoreMesh(core_axis_name="c", subcore_axis_name="s")


def _leaky(x):
  return jnp.where(x >= 0, x, 0.1 * x)


def _place16(s):
  r = lax.broadcasted_iota(jnp.int32, (16, 128), 0)
  p = lax.broadcasted_iota(jnp.int32, (16, 128), 1)
  return (p == s * 16 + r).astype(jnp.float32)


def _sel16(s):
  p = lax.broadcasted_iota(jnp.int32, (128, 16), 0)
  c = lax.broadcasted_iota(jnp.int32, (128, 16), 1)
  return (p == s * 16 + c).astype(jnp.float32)


def _node_mlp_body(x3_ref, w1_ref, b1_ref, w2p_ref, b2_ref, o_ref):
  acc = None
  for s in range(8):
    h = jnp.dot(x3_ref[:, s, :], w1_ref[...],
                preferred_element_type=jnp.float32)
    h = _leaky(h + b1_ref[...])
    y = jnp.dot(h, w2p_ref[...], preferred_element_type=jnp.float32)
    y = jnp.dot(y, _place16(s), preferred_element_type=jnp.float32)
    acc = y if acc is None else acc + y
  o_ref[...] = acc + b2_ref[...]


def _node_mlp(x3, Wn1, bn1, Wn2p, bn2r, npad, blkg):
  ng = npad // 8
  grid = ng // blkg
  return pl.pallas_call(
      _node_mlp_body,
      grid=(grid,),
      in_specs=[
          pl.BlockSpec((blkg, 8, 128), lambda i: (i, 0, 0)),
          pl.BlockSpec((128, 128), lambda i: (0, 0)),
          pl.BlockSpec((1, 128), lambda i: (0, 0)),
          pl.BlockSpec((128, 16), lambda i: (0, 0)),
          pl.BlockSpec((1, 128), lambda i: (0, 0)),
      ],
      out_specs=pl.BlockSpec((blkg, 128), lambda i: (i, 0)),
      out_shape=jax.ShapeDtypeStruct((ng, 128), jnp.float32),
  )(x3, Wn1, bn1, Wn2p, bn2r)


def _make_first3_local(E, npad):
  ew = E // NW
  tcols = E // 128
  ntl = ew // 128 + 2

  def body(ei_hbm, cnt_hbm, slots_hbm, ei_v, cnt_v, slots_v):
    cid = lax.axis_index("c")
    sid = lax.axis_index("s")
    w = sid * 2 + cid
    a = w * ew
    ta = jnp.minimum(a // 128, tcols - ntl)
    pltpu.sync_copy(ei_hbm.at[pl.ds(ta * 256, ntl * 256)], ei_v)

    def zero_body(i, _):
      cnt_v[pl.ds(i * 16, 16)] = jnp.zeros((16,), jnp.int32)
      return 0
    lax.fori_loop(0, npad // 16, zero_body, 0)

    iota = lax.iota(jnp.int32, 16)

    def scan_body(kv, _):
      pos = kv * 16 + (kv // 8) * 128
      s = ei_v[pl.ds(pos, 16)]
      eid = ta * 128 + kv * 16 + iota
      valid = (eid >= a) & (eid < a + ew)
      cnt1, last = plsc.scan_count(s, mask=valid)
      prior = plsc.load_gather(cnt_v, [s])
      r = prior + cnt1 - 1
      slot = s * 3 + jnp.minimum(jnp.maximum(r, 0), 2)
      plsc.store_scatter(slots_v, [slot], eid, mask=valid & (r < 3))
      plsc.store_scatter(cnt_v, [s], prior + cnt1, mask=last & valid)
      return 0
    lax.fori_loop(0, ntl * 8, scan_body, 0)

    pltpu.sync_copy(cnt_v, cnt_hbm.at[w])
    pltpu.sync_copy(slots_v, slots_hbm.at[w])

  return pl.kernel(
      body,
      out_type=(jax.ShapeDtypeStruct((NW, npad), jnp.int32),
                jax.ShapeDtypeStruct((NW, 3 * npad), jnp.int32)),
      mesh=_SC_MESH,
      compiler_params=_SC_PARAMS,
      scratch_types=[
          pltpu.VMEM((ntl * 256,), jnp.int32),
          pltpu.VMEM((npad,), jnp.int32),
          pltpu.VMEM((3 * npad,), jnp.int32),
      ])


def _make_merge_gather(E, npad):
  npw = npad // NW
  tcols = E // 128

  def body(cnt_hbm, slots_hbm, eif_hbm, ea_hbm, nf_hbm,
           packed_out, cnt_out,
           loc_cnt, loc_slots, cm_v, m0_v, m1_v, m2_v, nb3_v, rows3_v,
           pk_v, idx_v, ev_v, sem, sem2):
    cid = lax.axis_index("c")
    sid = lax.axis_index("s")
    wid = sid * 2 + cid
    nbase = wid * npw

    c1 = pltpu.async_copy(cnt_hbm.at[:, pl.ds(nbase, npw)], loc_cnt, sem)
    c2 = pltpu.async_copy(slots_hbm.at[:, pl.ds(3 * nbase, 3 * npw)],
                          loc_slots, sem)
    c1.wait()
    c2.wait()

    iota = lax.iota(jnp.int32, 16)
    zero = jnp.zeros((16,), jnp.int32)

    def merge_body(nv, _):
      nloc3 = (nv * 16 + iota) * 3

      def m_cond(st):
        w, cnt = st[0], st[1]
        return (w < NW) & (jnp.min(cnt) < 3)

      def m_step(st):
        w, cnt, mm0, mm1, mm2 = st
        c = loc_cnt[w, pl.ds(nv * 16, 16)]
        wv = jnp.full((16,), 0, jnp.int32) + w
        b0 = plsc.load_gather(loc_slots, [wv, nloc3])
        b1 = plsc.load_gather(loc_slots, [wv, nloc3 + 1])
        b2 = plsc.load_gather(loc_slots, [wv, nloc3 + 2])
        ce = jnp.minimum(c, 3)
        f0 = cnt == 0
        f1 = cnt == 1
        f2 = cnt == 2
        mm0 = jnp.where(f0 & (ce >= 1), b0, mm0)
        mm1 = jnp.where(f0 & (ce >= 2), b1,
                        jnp.where(f1 & (ce >= 1), b0, mm1))
        mm2 = jnp.where(f0 & (ce >= 3), b2,
                        jnp.where(f1 & (ce >= 2), b1,
                                  jnp.where(f2 & (ce >= 1), b0, mm2)))
        return (w + 1, jnp.minimum(cnt + ce, 3), mm0, mm1, mm2)

      _, cnt, mm0, mm1, mm2 = lax.while_loop(
          m_cond, m_step, (jnp.int32(0), zero, zero, zero, zero))
      fb = nbase + nv * 16 + iota
      sl = pl.ds(nv * 16, 16)
      m0_v[sl] = jnp.where(cnt >= 1, mm0, fb)
      m1_v[sl] = jnp.where(cnt >= 2, mm1, fb)
      m2_v[sl] = jnp.where(cnt >= 3, mm2, fb)
      cm_v[sl] = cnt
      return 0
    lax.fori_loop(0, npw // 16, merge_body, 0)

    zeros16f = jnp.zeros((16,), jnp.float32)
    mvs = (m0_v, m1_v, m2_v)

    own = pltpu.async_copy(nf_hbm.at[pl.ds(nbase, npw)],
                           rows3_v.at[pl.ds(0, npw)], sem)

    for j, m_v in enumerate(mvs):
      def gidx_body(t, _):
        m = m_v[pl.ds(t * 16, 16)]
        g = lax.shift_right_logical(m, 7) * 1024 + (m & 127)
        for c in range(16):
          fc = (c // 8) * (tcols * 1024) + (c % 8) * 128
          idx_v[pl.ds((j * 16 + c) * npw + t * 16, 16)] = g + fc
        return 0
      lax.fori_loop(0, npw // 16, gidx_body, 0)
    ea_cp = pltpu.async_copy(ea_hbm.at[idx_v], ev_v, sem2)

    d_cps = []
    for j, m_v in enumerate(mvs):
      def didx_body(t, _):
        m = m_v[pl.ds(t * 16, 16)]
        m_v[pl.ds(t * 16, 16)] = (
            lax.shift_right_logical(m, 7) * 256 + 128 + (m & 127))
        return 0
      lax.fori_loop(0, npw // 16, didx_body, 0)
      d_cps.append(pltpu.async_copy(eif_hbm.at[m_v],
                                    nb3_v.at[pl.ds(j * npw, npw)], sem))

    own.wait()

    def own_body(t, _):
      pk_v[t, pl.ds(96, 16)] = rows3_v[t, pl.ds(0, 16)]
      pk_v[t, pl.ds(112, 16)] = zeros16f
      return 0
    lax.fori_loop(0, npw, own_body, 0)

    n_cps = []
    for j in range(3):
      d_cps[j].wait()
      n_cps.append(pltpu.async_copy(
          nf_hbm.at[nb3_v.at[pl.ds(j * npw, npw)]],
          rows3_v.at[pl.ds(j * npw, npw)], sem))

    ea_cp.wait()
    for j in range(3):
      def esc_body(t, _):
        rows = t * 16 + iota
        for c in range(16):
          v = ev_v[pl.ds((j * 16 + c) * npw + t * 16, 16)]
          plsc.store_scatter(pk_v, [rows, jnp.full((16,), j * 16 + c,
                                                   jnp.int32)], v)
        return 0
      lax.fori_loop(0, npw // 16, esc_body, 0)

    for j in range(3):
      n_cps[j].wait()

      def nfc_body(t, _):
        pk_v[t, pl.ds(48 + j * 16, 16)] = rows3_v[j * npw + t, pl.ds(0, 16)]
        return 0
      lax.fori_loop(0, npw, nfc_body, 0)

    pltpu.sync_copy(pk_v, packed_out.at[pl.ds(nbase, npw)])
    pltpu.sync_copy(cm_v, cnt_out.at[pl.ds(nbase, npw)])

  return pl.kernel(
      body,
      out_type=(jax.ShapeDtypeStruct((npad, 128), jnp.float32),
                jax.ShapeDtypeStruct((npad,), jnp.int32)),
      mesh=_SC_MESH,
      compiler_params=_SC_PARAMS,
      scratch_types=[
          pltpu.VMEM((NW, npw), jnp.int32),
          pltpu.VMEM((NW, 3 * npw), jnp.int32),
          pltpu.VMEM((npw,), jnp.int32),
          pltpu.VMEM((npw,), jnp.int32),
          pltpu.VMEM((npw,), jnp.int32),
          pltpu.VMEM((npw,), jnp.int32),
          pltpu.VMEM((3 * npw,), jnp.int32),
          pltpu.VMEM((3 * npw, 16), jnp.float32),
          pltpu.VMEM((npw, 128), jnp.float32),
          pltpu.VMEM((48 * npw,), jnp.int32),
          pltpu.VMEM((48 * npw,), jnp.float32),
          pltpu.SemaphoreType.DMA,
          pltpu.SemaphoreType.DMA,
      ])


def _tail_body(pk_ref, cnt_ref, bat_ref,
               We1_ref, be1_ref, We2p_ref, be2p_ref,
               Wm1ap_ref, Wm1bp_ref, bm1_ref, Wm2p_ref, bm2p3_ref,
               Wu1ap_ref, Wu1bp_ref, bu1_ref, Wu2p_ref, bu2p_ref,
               Wh1p_ref, bh1_ref, Wh2_ref, bh2_ref,
               o_ref, acc_ref):
  i = pl.program_id(0)
  n = pl.num_programs(0)

  pk = pk_ref[...]
  s_msg = None
  for j in range(3):
    e_j = jnp.dot(pk, _sel16(j), preferred_element_type=jnp.float32)
    nf_j = jnp.dot(pk, _sel16(3 + j), preferred_element_type=jnp.float32)
    he = _leaky(jnp.dot(e_j, We1_ref[...],
                        preferred_element_type=jnp.float32) + be1_ref[...])
    ef = jnp.dot(he, We2p_ref[...],
                 preferred_element_type=jnp.float32) + be2p_ref[...]
    pre = (jnp.dot(ef, Wm1ap_ref[...], preferred_element_type=jnp.float32)
           + jnp.dot(nf_j, Wm1bp_ref[...],
                     preferred_element_type=jnp.float32) + bm1_ref[...])
    lj = _leaky(pre)
    s_msg = lj if s_msg is None else s_msg + lj
  aggr = jnp.dot(s_msg, Wm2p_ref[...],
                 preferred_element_type=jnp.float32) + bm2p3_ref[...]

  nf0 = jnp.dot(pk, _sel16(6), preferred_element_type=jnp.float32)
  hu = _leaky(jnp.dot(nf0, Wu1ap_ref[...], preferred_element_type=jnp.float32)
              + jnp.dot(aggr, Wu1bp_ref[...],
                        preferred_element_type=jnp.float32) + bu1_ref[...])
  nc = jnp.dot(hu, Wu2p_ref[...],
               preferred_element_type=jnp.float32) + bu2p_ref[...]
  valid = cnt_ref[...] >= 3
  nf_final = nf0 + jnp.where(valid, nc, 0.0)

  gids = lax.broadcasted_iota(jnp.int32, (1, NUM_GRAPHS), 1)
  oh = (bat_ref[...] == gids).astype(jnp.float32)
  seg = lax.dot_general(oh, nf_final, (((0,), (0,)), ((), ())),
                        preferred_element_type=jnp.float32)

  @pl.when(i == 0)
  def _():
    acc_ref[...] = jnp.zeros_like(acc_ref)
  acc_ref[...] += seg

  @pl.when(i == n - 1)
  def _():
    hh = _leaky(jnp.dot(acc_ref[...], Wh1p_ref[...],
                        preferred_element_type=jnp.float32) + bh1_ref[...])
    o_ref[...] = jnp.dot(hh, Wh2_ref[...],
                         preferred_element_type=jnp.float32) + bh2_ref[...]


def _tail(packed, cnt2d, bat2d, weights, npad, blk):
  grid = npad // blk
  full = lambda shape: pl.BlockSpec(shape, lambda i: tuple(0 for _ in shape))
  in_specs = [
      pl.BlockSpec((blk, 128), lambda i: (i, 0)),
      pl.BlockSpec((blk, 1), lambda i: (i, 0)),
      pl.BlockSpec((blk, 1), lambda i: (i, 0)),
      full((16, 128)), full((1, 128)), full((128, 16)), full((1, 16)),
      full((16, 128)), full((16, 128)), full((1, 128)),
      full((128, 16)), full((1, 16)),
      full((16, 128)), full((16, 128)), full((1, 128)),
      full((128, 16)), full((1, 16)),
      full((16, 128)), full((1, 128)), full((128, 2)), full((1, 2)),
  ]
  return pl.pallas_call(
      _tail_body,
      grid=(grid,),
      in_specs=in_specs,
      out_specs=pl.BlockSpec((NUM_GRAPHS, 2), lambda i: (0, 0)),
      out_shape=jax.ShapeDtypeStruct((NUM_GRAPHS, 2), jnp.float32),
      scratch_shapes=[pltpu.VMEM((NUM_GRAPHS, 16), jnp.float32)],
  )(packed, cnt2d, bat2d, *weights)


def kernel(node_feat, edge_attr, edge_index, batch,
           Wn1, bn1, Wn2, bn2, We1, be1, We2, be2,
           Wm1, bm1, Wm2, bm2, Wu1, bu1, Wu2, bu2,
           Wh1, bh1, Wh2, bh2):
  N, DF = node_feat.shape
  E = edge_attr.shape[0]
  npad = ((N + NW * 16 - 1) // (NW * 16)) * (NW * 16)
  blk = 2048
  tcols = E // 128

  node_feat_pad = jnp.pad(node_feat.astype(jnp.float32),
                          ((0, npad - N), (0, 0)))
  x3 = node_feat_pad.reshape(npad // 8, 8, 128)
  bat2d = jnp.pad(batch.astype(jnp.int32), (0, npad - N),
                  constant_values=NUM_GRAPHS).reshape(npad, 1)

  ea_lin = (edge_attr.astype(jnp.float32).T
            .reshape(2, 8, tcols, 128)
            .transpose(0, 2, 1, 3)
            .reshape(E * 16))
  ei_flat = edge_index.reshape(2, tcols, 128).transpose(1, 0, 2).reshape(2 * E)

  f32 = jnp.float32
  Wn2p = jnp.zeros((128, 16), f32).at[:, :3].set(Wn2)
  bn2r = jnp.tile(jnp.zeros((16,), f32).at[:3].set(bn2), 8).reshape(1, 128)
  We2p = jnp.zeros((128, 16), f32).at[:, :3].set(We2)
  be2p = jnp.zeros((1, 16), f32).at[0, :3].set(be2)
  Wm1ap = jnp.zeros((16, 128), f32).at[:3].set(Wm1[:3])
  Wm1bp = jnp.zeros((16, 128), f32).at[:3].set(Wm1[3:6])
  Wm2p = jnp.zeros((128, 16), f32).at[:, :2].set(Wm2)
  bm2p3 = jnp.zeros((1, 16), f32).at[0, :2].set(3.0 * bm2)
  Wu1ap = jnp.zeros((16, 128), f32).at[:3].set(Wu1[:3])
  Wu1bp = jnp.zeros((16, 128), f32).at[:2].set(Wu1[3:5])
  Wu2p = jnp.zeros((128, 16), f32).at[:, :3].set(Wu2)
  bu2p = jnp.zeros((1, 16), f32).at[0, :3].set(bu2)
  Wh1p = jnp.zeros((16, 128), f32).at[:3].set(Wh1)

  nf128 = _node_mlp(x3, Wn1, bn1.reshape(1, 128), Wn2p, bn2r, npad,
                    npad // 8 // 5)
  nf_lin = nf128.reshape(npad, 16)

  cnt_loc, slots_loc = _make_first3_local(E, npad)(ei_flat)
  packed, cntm = _make_merge_gather(E, npad)(
      cnt_loc, slots_loc, ei_flat, ea_lin, nf_lin)

  weights = (We1, be1.reshape(1, 128), We2p, be2p,
             Wm1ap, Wm1bp, bm1.reshape(1, 128), Wm2p, bm2p3,
             Wu1ap, Wu1bp, bu1.reshape(1, 128), Wu2p, bu2p,
             Wh1p, bh1.reshape(1, 128), Wh2, bh2.reshape(1, 2))
  return _tail(packed, cntm.reshape(npad, 1), bat2d, weights, npad, blk)


# speedup vs baseline: 1.0411x; 1.0139x over previous
import functools
import jax
import jax.numpy as jnp
from jax import lax
from jax.experimental import pallas as pl
from jax.experimental.pallas import tpu as pltpu, tpu_sc as plsc

NUM_GRAPHS = 16
NW = 32

_SC_PARAMS = pltpu.CompilerParams(
    needs_layout_passes=False, use_tc_tiling_on_sc=False)
_SC_MESH = plsc.VectorSubcoreMesh(core_axis_name="c", subcore_axis_name="s")


def _leaky(x):
  return jnp.where(x >= 0, x, 0.1 * x)


def _place16(s):
  r = lax.broadcasted_iota(jnp.int32, (16, 128), 0)
  p = lax.broadcasted_iota(jnp.int32, (16, 128), 1)
  return (p == s * 16 + r).astype(jnp.float32)


def _sel16(s):
  p = lax.broadcasted_iota(jnp.int32, (128, 16), 0)
  c = lax.broadcasted_iota(jnp.int32, (128, 16), 1)
  return (p == s * 16 + c).astype(jnp.float32)


def _node_mlp_body(x3_ref, w1_ref, b1_ref, w2p_ref, b2_ref, o_ref):
  acc = None
  for s in range(8):
    h = jnp.dot(x3_ref[:, s, :], w1_ref[...],
                preferred_element_type=jnp.float32)
    h = _leaky(h + b1_ref[...])
    y = jnp.dot(h, w2p_ref[...], preferred_element_type=jnp.float32)
    y = jnp.dot(y, _place16(s), preferred_element_type=jnp.float32)
    acc = y if acc is None else acc + y
  o_ref[...] = acc + b2_ref[...]


def _node_mlp(x3, Wn1, bn1, Wn2p, bn2r, npad, blkg):
  ng = npad // 8
  grid = ng // blkg
  return pl.pallas_call(
      _node_mlp_body,
      grid=(grid,),
      in_specs=[
          pl.BlockSpec((blkg, 8, 128), lambda i: (i, 0, 0)),
          pl.BlockSpec((128, 128), lambda i: (0, 0)),
          pl.BlockSpec((1, 128), lambda i: (0, 0)),
          pl.BlockSpec((128, 16), lambda i: (0, 0)),
          pl.BlockSpec((1, 128), lambda i: (0, 0)),
      ],
      out_specs=pl.BlockSpec((blkg, 128), lambda i: (i, 0)),
      out_shape=jax.ShapeDtypeStruct((ng, 128), jnp.float32),
  )(x3, Wn1, bn1, Wn2p, bn2r)


def _make_first3_local(E, npad):
  ew = E // NW
  tcols = E // 128
  ntl = ew // 128 + 2

  def body(ei_hbm, cnt_hbm, slots_hbm, ei_v, cnt_v, slots_v):
    cid = lax.axis_index("c")
    sid = lax.axis_index("s")
    w = sid * 2 + cid
    a = w * ew
    ta = jnp.minimum(a // 128, tcols - ntl)
    pltpu.sync_copy(ei_hbm.at[pl.ds(ta * 256, ntl * 256)], ei_v)

    def zero_body(i, _):
      cnt_v[pl.ds(i * 16, 16)] = jnp.zeros((16,), jnp.int32)
      return 0
    lax.fori_loop(0, npad // 16, zero_body, 0)

    iota = lax.iota(jnp.int32, 16)

    def scan_body(kv, _):
      pos = kv * 16 + (kv // 8) * 128
      s = ei_v[pl.ds(pos, 16)]
      eid = ta * 128 + kv * 16 + iota
      valid = (eid >= a) & (eid < a + ew)
      cnt1, last = plsc.scan_count(s, mask=valid)
      prior = plsc.load_gather(cnt_v, [s])
      r = prior + cnt1 - 1
      slot = s * 3 + jnp.minimum(jnp.maximum(r, 0), 2)
      plsc.store_scatter(slots_v, [slot], eid, mask=valid & (r < 3))
      plsc.store_scatter(cnt_v, [s], prior + cnt1, mask=last & valid)
      return 0
    lax.fori_loop(0, ntl * 8, scan_body, 0)

    pltpu.sync_copy(cnt_v, cnt_hbm.at[w])
    pltpu.sync_copy(slots_v, slots_hbm.at[w])

  return pl.kernel(
      body,
      out_type=(jax.ShapeDtypeStruct((NW, npad), jnp.int32),
                jax.ShapeDtypeStruct((NW, 3 * npad), jnp.int32)),
      mesh=_SC_MESH,
      compiler_params=_SC_PARAMS,
      scratch_types=[
          pltpu.VMEM((ntl * 256,), jnp.int32),
          pltpu.VMEM((npad,), jnp.int32),
          pltpu.VMEM((3 * npad,), jnp.int32),
      ])


def _make_merge_gather(E, npad):
  npw = npad // NW
  tcols = E // 128

  def body(cnt_hbm, slots_hbm, eif_hbm, ea_hbm, nf_hbm,
           packed_out, cnt_out,
           loc_cnt, loc_slots, cm_v, m0_v, m1_v, m2_v, nb3_v, rows3_v,
           pk_v, idx_v, ev_v, sem, sem2):
    cid = lax.axis_index("c")
    sid = lax.axis_index("s")
    wid = sid * 2 + cid
    nbase = wid * npw

    c1 = pltpu.async_copy(cnt_hbm.at[:, pl.ds(nbase, npw)], loc_cnt, sem)
    c2 = pltpu.async_copy(slots_hbm.at[:, pl.ds(3 * nbase, 3 * npw)],
                          loc_slots, sem)
    c1.wait()
    c2.wait()

    iota = lax.iota(jnp.int32, 16)
    zero = jnp.zeros((16,), jnp.int32)

    def merge_body(nv, _):
      nloc3 = (nv * 16 + iota) * 3

      def m_cond(st):
        w, cnt = st[0], st[1]
        return (w < NW) & (jnp.min(cnt) < 3)

      def m_one(w, cnt, mm0, mm1, mm2):
        c = loc_cnt[w, pl.ds(nv * 16, 16)]
        wv = jnp.full((16,), 0, jnp.int32) + w
        b0 = plsc.load_gather(loc_slots, [wv, nloc3])
        b1 = plsc.load_gather(loc_slots, [wv, nloc3 + 1])
        b2 = plsc.load_gather(loc_slots, [wv, nloc3 + 2])
        ce = jnp.minimum(c, 3)
        f0 = cnt == 0
        f1 = cnt == 1
        f2 = cnt == 2
        mm0 = jnp.where(f0 & (ce >= 1), b0, mm0)
        mm1 = jnp.where(f0 & (ce >= 2), b1,
                        jnp.where(f1 & (ce >= 1), b0, mm1))
        mm2 = jnp.where(f0 & (ce >= 3), b2,
                        jnp.where(f1 & (ce >= 2), b1,
                                  jnp.where(f2 & (ce >= 1), b0, mm2)))
        return jnp.minimum(cnt + ce, 3), mm0, mm1, mm2

      def m_step(st):
        w, cnt, mm0, mm1, mm2 = st
        cnt, mm0, mm1, mm2 = m_one(w, cnt, mm0, mm1, mm2)
        cnt, mm0, mm1, mm2 = m_one(w + 1, cnt, mm0, mm1, mm2)
        return (w + 2, cnt, mm0, mm1, mm2)

      _, cnt, mm0, mm1, mm2 = lax.while_loop(
          m_cond, m_step, (jnp.int32(0), zero, zero, zero, zero))
      fb = nbase + nv * 16 + iota
      sl = pl.ds(nv * 16, 16)
      m0_v[sl] = jnp.where(cnt >= 1, mm0, fb)
      m1_v[sl] = jnp.where(cnt >= 2, mm1, fb)
      m2_v[sl] = jnp.where(cnt >= 3, mm2, fb)
      cm_v[sl] = cnt
      return 0
    lax.fori_loop(0, npw // 16, merge_body, 0)

    zeros16f = jnp.zeros((16,), jnp.float32)
    mvs = (m0_v, m1_v, m2_v)

    own = pltpu.async_copy(nf_hbm.at[pl.ds(nbase, npw)],
                           rows3_v.at[pl.ds(0, npw)], sem)

    for j, m_v in enumerate(mvs):
      def gidx_body(t, _):
        m = m_v[pl.ds(t * 16, 16)]
        g = lax.shift_right_logical(m, 7) * 1024 + (m & 127)
        for c in range(16):
          fc = (c // 8) * (tcols * 1024) + (c % 8) * 128
          idx_v[pl.ds((j * 16 + c) * npw + t * 16, 16)] = g + fc
        return 0
      lax.fori_loop(0, npw // 16, gidx_body, 0)
    ea_cp = pltpu.async_copy(ea_hbm.at[idx_v], ev_v, sem2)

    d_cps = []
    for j, m_v in enumerate(mvs):
      def didx_body(t, _):
        m = m_v[pl.ds(t * 16, 16)]
        m_v[pl.ds(t * 16, 16)] = (
            lax.shift_right_logical(m, 7) * 256 + 128 + (m & 127))
        return 0
      lax.fori_loop(0, npw // 16, didx_body, 0)
      d_cps.append(pltpu.async_copy(eif_hbm.at[m_v],
                                    nb3_v.at[pl.ds(j * npw, npw)], sem))

    own.wait()

    def own_body(t8, _):
      for u in range(8):
        t = t8 * 8 + u
        pk_v[t, pl.ds(96, 16)] = rows3_v[t, pl.ds(0, 16)]
        pk_v[t, pl.ds(112, 16)] = zeros16f
      return 0
    lax.fori_loop(0, npw // 8, own_body, 0)

    n_cps = []
    for j in range(3):
      d_cps[j].wait()
      n_cps.append(pltpu.async_copy(
          nf_hbm.at[nb3_v.at[pl.ds(j * npw, npw)]],
          rows3_v.at[pl.ds(j * npw, npw)], sem))

    ea_cp.wait()
    for j in range(3):
      def esc_body(t, _):
        rows = t * 16 + iota
        for c in range(16):
          v = ev_v[pl.ds((j * 16 + c) * npw + t * 16, 16)]
          plsc.store_scatter(pk_v, [rows, jnp.full((16,), j * 16 + c,
                                                   jnp.int32)], v)
        return 0
      lax.fori_loop(0, npw // 16, esc_body, 0)

    for j in range(3):
      n_cps[j].wait()

      def nfc_body(t8, _):
        for u in range(8):
          t = t8 * 8 + u
          pk_v[t, pl.ds(48 + j * 16, 16)] = rows3_v[j * npw + t, pl.ds(0, 16)]
        return 0
      lax.fori_loop(0, npw // 8, nfc_body, 0)

    pltpu.sync_copy(pk_v, packed_out.at[pl.ds(nbase, npw)])
    pltpu.sync_copy(cm_v, cnt_out.at[pl.ds(nbase, npw)])

  return pl.kernel(
      body,
      out_type=(jax.ShapeDtypeStruct((npad, 128), jnp.float32),
                jax.ShapeDtypeStruct((npad,), jnp.int32)),
      mesh=_SC_MESH,
      compiler_params=_SC_PARAMS,
      scratch_types=[
          pltpu.VMEM((NW, npw), jnp.int32),
          pltpu.VMEM((NW, 3 * npw), jnp.int32),
          pltpu.VMEM((npw,), jnp.int32),
          pltpu.VMEM((npw,), jnp.int32),
          pltpu.VMEM((npw,), jnp.int32),
          pltpu.VMEM((npw,), jnp.int32),
          pltpu.VMEM((3 * npw,), jnp.int32),
          pltpu.VMEM((3 * npw, 16), jnp.float32),
          pltpu.VMEM((npw, 128), jnp.float32),
          pltpu.VMEM((48 * npw,), jnp.int32),
          pltpu.VMEM((48 * npw,), jnp.float32),
          pltpu.SemaphoreType.DMA,
          pltpu.SemaphoreType.DMA,
      ])


def _tail_body(pk_ref, cnt_ref, bat_ref,
               We1_ref, be1_ref, We2p_ref, be2p_ref,
               Wm1ap_ref, Wm1bp_ref, bm1_ref, Wm2p_ref, bm2p3_ref,
               Wu1ap_ref, Wu1bp_ref, bu1_ref, Wu2p_ref, bu2p_ref,
               Wh1p_ref, bh1_ref, Wh2_ref, bh2_ref,
               o_ref, acc_ref):
  i = pl.program_id(0)
  n = pl.num_programs(0)

  pk = pk_ref[...]
  s_msg = None
  for j in range(3):
    e_j = jnp.dot(pk, _sel16(j), preferred_element_type=jnp.float32)
    nf_j = jnp.dot(pk, _sel16(3 + j), preferred_element_type=jnp.float32)
    he = _leaky(jnp.dot(e_j, We1_ref[...],
                        preferred_element_type=jnp.float32) + be1_ref[...])
    ef = jnp.dot(he, We2p_ref[...],
                 preferred_element_type=jnp.float32) + be2p_ref[...]
    pre = (jnp.dot(ef, Wm1ap_ref[...], preferred_element_type=jnp.float32)
           + jnp.dot(nf_j, Wm1bp_ref[...],
                     preferred_element_type=jnp.float32) + bm1_ref[...])
    lj = _leaky(pre)
    s_msg = lj if s_msg is None else s_msg + lj
  aggr = jnp.dot(s_msg, Wm2p_ref[...],
                 preferred_element_type=jnp.float32) + bm2p3_ref[...]

  nf0 = jnp.dot(pk, _sel16(6), preferred_element_type=jnp.float32)
  hu = _leaky(jnp.dot(nf0, Wu1ap_ref[...], preferred_element_type=jnp.float32)
              + jnp.dot(aggr, Wu1bp_ref[...],
                        preferred_element_type=jnp.float32) + bu1_ref[...])
  nc = jnp.dot(hu, Wu2p_ref[...],
               preferred_element_type=jnp.float32) + bu2p_ref[...]
  valid = cnt_ref[...] >= 3
  nf_final = nf0 + jnp.where(valid, nc, 0.0)

  gids = lax.broadcasted_iota(jnp.int32, (1, NUM_GRAPHS), 1)
  oh = (bat_ref[...] == gids).astype(jnp.float32)
  seg = lax.dot_general(oh, nf_final, (((0,), (0,)), ((), ())),
                        preferred_element_type=jnp.float32)

  @pl.when(i == 0)
  def _():
    acc_ref[...] = jnp.zeros_like(acc_ref)
  acc_ref[...] += seg

  @pl.when(i == n - 1)
  def _():
    hh = _leaky(jnp.dot(acc_ref[...], Wh1p_ref[...],
                        preferred_element_type=jnp.float32) + bh1_ref[...])
    o_ref[...] = jnp.dot(hh, Wh2_ref[...],
                         preferred_element_type=jnp.float32) + bh2_ref[...]


def _tail(packed, cnt2d, bat2d, weights, npad, blk):
  grid = npad // blk
  full = lambda shape: pl.BlockSpec(shape, lambda i: tuple(0 for _ in shape))
  in_specs = [
      pl.BlockSpec((blk, 128), lambda i: (i, 0)),
      pl.BlockSpec((blk, 1), lambda i: (i, 0)),
      pl.BlockSpec((blk, 1), lambda i: (i, 0)),
      full((16, 128)), full((1, 128)), full((128, 16)), full((1, 16)),
      full((16, 128)), full((16, 128)), full((1, 128)),
      full((128, 16)), full((1, 16)),
      full((16, 128)), full((16, 128)), full((1, 128)),
      full((128, 16)), full((1, 16)),
      full((16, 128)), full((1, 128)), full((128, 2)), full((1, 2)),
  ]
  return pl.pallas_call(
      _tail_body,
      grid=(grid,),
      in_specs=in_specs,
      out_specs=pl.BlockSpec((NUM_GRAPHS, 2), lambda i: (0, 0)),
      out_shape=jax.ShapeDtypeStruct((NUM_GRAPHS, 2), jnp.float32),
      scratch_shapes=[pltpu.VMEM((NUM_GRAPHS, 16), jnp.float32)],
  )(packed, cnt2d, bat2d, *weights)


def kernel(node_feat, edge_attr, edge_index, batch,
           Wn1, bn1, Wn2, bn2, We1, be1, We2, be2,
           Wm1, bm1, Wm2, bm2, Wu1, bu1, Wu2, bu2,
           Wh1, bh1, Wh2, bh2):
  N, DF = node_feat.shape
  E = edge_attr.shape[0]
  npad = ((N + NW * 16 - 1) // (NW * 16)) * (NW * 16)
  blk = npad
  tcols = E // 128

  node_feat_pad = jnp.pad(node_feat.astype(jnp.float32),
                          ((0, npad - N), (0, 0)))
  x3 = node_feat_pad.reshape(npad // 8, 8, 128)
  bat2d = jnp.pad(batch.astype(jnp.int32), (0, npad - N),
                  constant_values=NUM_GRAPHS).reshape(npad, 1)

  ea_lin = (edge_attr.astype(jnp.float32).T
            .reshape(2, 8, tcols, 128)
            .transpose(0, 2, 1, 3)
            .reshape(E * 16))
  ei_flat = edge_index.reshape(2, tcols, 128).transpose(1, 0, 2).reshape(2 * E)

  f32 = jnp.float32
  Wn2p = jnp.zeros((128, 16), f32).at[:, :3].set(Wn2)
  bn2r = jnp.tile(jnp.zeros((16,), f32).at[:3].set(bn2), 8).reshape(1, 128)
  We2p = jnp.zeros((128, 16), f32).at[:, :3].set(We2)
  be2p = jnp.zeros((1, 16), f32).at[0, :3].set(be2)
  Wm1ap = jnp.zeros((16, 128), f32).at[:3].set(Wm1[:3])
  Wm1bp = jnp.zeros((16, 128), f32).at[:3].set(Wm1[3:6])
  Wm2p = jnp.zeros((128, 16), f32).at[:, :2].set(Wm2)
  bm2p3 = jnp.zeros((1, 16), f32).at[0, :2].set(3.0 * bm2)
  Wu1ap = jnp.zeros((16, 128), f32).at[:3].set(Wu1[:3])
  Wu1bp = jnp.zeros((16, 128), f32).at[:2].set(Wu1[3:5])
  Wu2p = jnp.zeros((128, 16), f32).at[:, :3].set(Wu2)
  bu2p = jnp.zeros((1, 16), f32).at[0, :3].set(bu2)
  Wh1p = jnp.zeros((16, 128), f32).at[:3].set(Wh1)

  nf128 = _node_mlp(x3, Wn1, bn1.reshape(1, 128), Wn2p, bn2r, npad,
                    npad // 8)
  nf_lin = nf128.reshape(npad, 16)

  cnt_loc, slots_loc = _make_first3_local(E, npad)(ei_flat)
  packed, cntm = _make_merge_gather(E, npad)(
      cnt_loc, slots_loc, ei_flat, ea_lin, nf_lin)

  weights = (We1, be1.reshape(1, 128), We2p, be2p,
             Wm1ap, Wm1bp, bm1.reshape(1, 128), Wm2p, bm2p3,
             Wu1ap, Wu1bp, bu1.reshape(1, 128), Wu2p, bu2p,
             Wh1p, bh1.reshape(1, 128), Wh2, bh2.reshape(1, 2))
  return _tail(packed, cntm.reshape(npad, 1), bat2d, weights, npad, blk)
